# baseline (speedup 1.0000x reference)
import functools

import jax
import jax.numpy as jnp
from jax import lax
from jax.experimental import pallas as pl
from jax.experimental.pallas import tpu as pltpu
from jax.experimental.pallas import tpu_sc as plsc

N = 10000
E = 320000
D = 128
NB = 64

NC = 2
NS = 16
NW = NC * NS
K = 128
CH = 80
NPAIR = CH // 2
EPW = CH * K
EPAD = NW * EPW
SINK = 128
NA = N + SINK
RPS = 624
TAIL = N - NS * RPS
ZR = 208

R = 2000
GRID = N // R

DW = 16


@functools.cache
def _make_sc_edge_agg():
    mesh = plsc.VectorSubcoreMesh(core_axis_name="c", subcore_axis_name="s")

    @functools.partial(
        pl.kernel, mesh=mesh,
        out_type=jax.ShapeDtypeStruct((NC, N, D), jnp.float32),
        scratch_types=[
            pltpu.VMEM((K,), jnp.int32),
            pltpu.VMEM((K,), jnp.int32),
            pltpu.VMEM((K,), jnp.int32),
            pltpu.VMEM((K,), jnp.int32),
            pltpu.VMEM((K, D), jnp.float32),
            pltpu.VMEM((K, D), jnp.float32),
            pltpu.VMEM_SHARED((NA, D), jnp.float32),
            pltpu.SemaphoreType.DMA,
            pltpu.SemaphoreType.DMA,
        ],
    )
    def sc_edge_agg(hp_hbm, srcf_hbm, dstf_hbm, zrows_hbm, out_hbm,
                    src_a, src_b, dst_a, dst_b, rows_a, rows_b, acc,
                    gsem, isem):
        c = lax.axis_index("c")
        s = lax.axis_index("s")
        wid = c * NS + s
        base0 = wid * EPW
        for j in range(RPS // ZR):
            pltpu.sync_copy(zrows_hbm, acc.at[pl.ds(s * RPS + j * ZR, ZR)])

        @pl.when(s == 0)
        def _():
            pltpu.sync_copy(zrows_hbm.at[pl.ds(0, TAIL)],
                            acc.at[pl.ds(NS * RPS, TAIL)])
        plsc.subcore_barrier()

        pltpu.sync_copy(srcf_hbm.at[pl.ds(base0, K)], src_a)
        pltpu.sync_copy(dstf_hbm.at[pl.ds(base0, K)], dst_a)
        pltpu.async_copy(hp_hbm.at[src_a], rows_a, gsem)
        pltpu.async_copy(srcf_hbm.at[pl.ds(base0 + K, K)], src_b, isem)
        pltpu.async_copy(dstf_hbm.at[pl.ds(base0 + K, K)], dst_b, isem)

        def half(i, sA, dA, rA, sB, dB, rB):
            pltpu.make_async_copy(hp_hbm.at[sA], rA, gsem).wait()

            @pl.when(i + 1 < CH)
            def _():
                pltpu.make_async_copy(
                    srcf_hbm.at[pl.ds(base0, K)], sB, isem).wait()
                pltpu.make_async_copy(
                    dstf_hbm.at[pl.ds(base0, K)], dB, isem).wait()
                pltpu.async_copy(hp_hbm.at[sB], rB, gsem)

            pltpu.sync_copy(rA, acc.at[dA], add=True)

            @pl.when(i + 2 < CH)
            def _():
                pltpu.async_copy(
                    srcf_hbm.at[pl.ds(base0 + (i + 2) * K, K)], sA, isem)
                pltpu.async_copy(
                    dstf_hbm.at[pl.ds(base0 + (i + 2) * K, K)], dA, isem)

        def pair(j, carry):
            i0 = 2 * j
            half(i0, src_a, dst_a, rows_a, src_b, dst_b, rows_b)
            half(i0 + 1, src_b, dst_b, rows_b, src_a, dst_a, rows_a)
            return carry

        lax.fori_loop(0, NPAIR, pair, 0)
        plsc.subcore_barrier()
        pltpu.sync_copy(acc.at[pl.ds(s * RPS, RPS)],
                        out_hbm.at[c, pl.ds(s * RPS, RPS)])

        @pl.when(s == 0)
        def _():
            pltpu.sync_copy(acc.at[pl.ds(NS * RPS, TAIL)],
                            out_hbm.at[c, pl.ds(NS * RPS, TAIL)])

    return sc_edge_agg


def _sc_edge_agg(hp, srcf, dstf, zrows):
    return _make_sc_edge_agg()(hp, srcf, dstf, zrows)


@functools.cache
def _make_sc_deg():
    mesh = plsc.VectorSubcoreMesh(core_axis_name="c", subcore_axis_name="s")

    @functools.partial(
        pl.kernel, mesh=mesh,
        out_type=jax.ShapeDtypeStruct((NC, N, D), jnp.float32),
        scratch_types=[
            pltpu.VMEM((CH, K), jnp.int32),
            pltpu.VMEM((K, D), jnp.float32),
            pltpu.VMEM_SHARED((NA, D), jnp.float32),
        ],
    )
    def sc_deg(dst3_hbm, ones_hbm, zrows_hbm, out_hbm, dsts_v, ones_v, acc):
        c = lax.axis_index("c")
        s = lax.axis_index("s")
        wid = c * NS + s
        pltpu.sync_copy(dst3_hbm.at[wid], dsts_v)
        for j in range(RPS // ZR):
            pltpu.sync_copy(zrows_hbm, acc.at[pl.ds(s * RPS + j * ZR, ZR)])

        @pl.when(s == 0)
        def _():
            pltpu.sync_copy(zrows_hbm.at[pl.ds(0, TAIL)],
                            acc.at[pl.ds(NS * RPS, TAIL)])
        pltpu.sync_copy(ones_hbm, ones_v)
        plsc.subcore_barrier()

        def step(i, carry):
            pltpu.sync_copy(ones_v, acc.at[dsts_v.at[i]], add=True)
            return carry

        lax.fori_loop(0, CH, step, 0)
        plsc.subcore_barrier()
        pltpu.sync_copy(acc.at[pl.ds(s * RPS, RPS)],
                        out_hbm.at[c, pl.ds(s * RPS, RPS)])

        @pl.when(s == 0)
        def _():
            pltpu.sync_copy(acc.at[pl.ds(NS * RPS, TAIL)],
                            out_hbm.at[c, pl.ds(NS * RPS, TAIL)])

    return sc_deg


def _sc_deg(dst3, onesr, zrows):
    return _make_sc_deg()(dst3, onesr, zrows)


def _lead_body(x_ref, w_ref, b_ref, degp_ref, h_ref, hp_ref, dis_ref):
    deg = degp_ref[0, :, 0:1] + degp_ref[1, :, 0:1] + 1.0
    dis = lax.rsqrt(deg)
    h = jnp.dot(x_ref[...], w_ref[...], preferred_element_type=jnp.float32)
    h = h + b_ref[...]
    h_ref[...] = h
    hp_ref[...] = h * dis
    dis_ref[...] = dis


def _tc_lead(x, w, b, degp):
    return pl.pallas_call(
        _lead_body,
        grid=(GRID,),
        in_specs=[
            pl.BlockSpec((R, D), lambda i: (i, 0)),
            pl.BlockSpec((D, D), lambda i: (0, 0)),
            pl.BlockSpec((1, D), lambda i: (0, 0)),
            pl.BlockSpec((NC, R, D), lambda i: (0, i, 0)),
        ],
        out_specs=[
            pl.BlockSpec((R, D), lambda i: (i, 0)),
            pl.BlockSpec((R, D), lambda i: (i, 0)),
            pl.BlockSpec((R, 1), lambda i: (i, 0)),
        ],
        out_shape=[
            jax.ShapeDtypeStruct((N, D), jnp.float32),
            jax.ShapeDtypeStruct((N, D), jnp.float32),
            jax.ShapeDtypeStruct((N, 1), jnp.float32),
        ],
    )(x, w, b, degp)


def _relu_stats_body(p_ref, h_ref, dis_ref, y_ref, ssum_ref, ssq_ref):
    dis = dis_ref[...]
    agg = dis * (p_ref[0] + p_ref[1]) + (dis * dis) * h_ref[...]
    y = jnp.maximum(agg, 0.0)
    y_ref[...] = y

    @pl.when(pl.program_id(0) == 0)
    def _():
        ssum_ref[...] = jnp.zeros_like(ssum_ref)
        ssq_ref[...] = jnp.zeros_like(ssq_ref)

    ssum_ref[...] += jnp.sum(y, axis=0, keepdims=True)
    ssq_ref[...] += jnp.sum(y * y, axis=0, keepdims=True)


def _tc_relu_stats(p, h, dis):
    return pl.pallas_call(
        _relu_stats_body,
        grid=(GRID,),
        in_specs=[
            pl.BlockSpec((NC, R, D), lambda i: (0, i, 0)),
            pl.BlockSpec((R, D), lambda i: (i, 0)),
            pl.BlockSpec((R, 1), lambda i: (i, 0)),
        ],
        out_specs=[
            pl.BlockSpec((R, D), lambda i: (i, 0)),
            pl.BlockSpec((1, D), lambda i: (0, 0)),
            pl.BlockSpec((1, D), lambda i: (0, 0)),
        ],
        out_shape=[
            jax.ShapeDtypeStruct((N, D), jnp.float32),
            jax.ShapeDtypeStruct((1, D), jnp.float32),
            jax.ShapeDtypeStruct((1, D), jnp.float32),
        ],
    )(p, h, dis)


def _bn_matmul_body(y_ref, ssum_ref, ssq_ref, g_ref, be_ref, w_ref, b_ref,
                    dis_ref, h_ref, hp_ref):
    mu = ssum_ref[...] / N
    var = ssq_ref[...] / N - mu * mu
    rstd = lax.rsqrt(var + 1e-5)
    xn = (y_ref[...] - mu) * (rstd * g_ref[...]) + be_ref[...]
    h = jnp.dot(xn, w_ref[...], preferred_element_type=jnp.float32)
    h = h + b_ref[...]
    h_ref[...] = h
    hp_ref[...] = h * dis_ref[...]


def _tc_bn_matmul(y, ssum, ssq, g, be, w, b, dis):
    return pl.pallas_call(
        _bn_matmul_body,
        grid=(GRID,),
        in_specs=[
            pl.BlockSpec((R, D), lambda i: (i, 0)),
            pl.BlockSpec((1, D), lambda i: (0, 0)),
            pl.BlockSpec((1, D), lambda i: (0, 0)),
            pl.BlockSpec((1, D), lambda i: (0, 0)),
            pl.BlockSpec((1, D), lambda i: (0, 0)),
            pl.BlockSpec((D, D), lambda i: (0, 0)),
            pl.BlockSpec((1, D), lambda i: (0, 0)),
            pl.BlockSpec((R, 1), lambda i: (i, 0)),
        ],
        out_specs=[
            pl.BlockSpec((R, D), lambda i: (i, 0)),
            pl.BlockSpec((R, D), lambda i: (i, 0)),
        ],
        out_shape=[
            jax.ShapeDtypeStruct((N, D), jnp.float32),
            jax.ShapeDtypeStruct((N, D), jnp.float32),
        ],
    )(y, ssum, ssq, g, be, w, b, dis)


def _bn_pool_body(y_ref, ssum_ref, ssq_ref, g_ref, be_ref, batch_ref,
                  segs_ref, cnt_ref):
    mu = ssum_ref[...] / N
    var = ssq_ref[...] / N - mu * mu
    rstd = lax.rsqrt(var + 1e-5)
    xn = (y_ref[...] - mu) * (rstd * g_ref[...]) + be_ref[...]
    seg_ids = lax.broadcasted_iota(jnp.int32, (1, NB), 1)
    oneh = jnp.where(batch_ref[...] == seg_ids, 1.0, 0.0)

    @pl.when(pl.program_id(0) == 0)
    def _():
        segs_ref[...] = jnp.zeros_like(segs_ref)
        cnt_ref[...] = jnp.zeros_like(cnt_ref)

    segs_ref[...] += lax.dot_general(oneh, xn, (((0,), (0,)), ((), ())),
                                     preferred_element_type=jnp.float32)
    ones_col = jnp.ones((R, 1), jnp.float32)
    cnt_ref[...] += lax.dot_general(oneh, ones_col, (((0,), (0,)), ((), ())),
                                    preferred_element_type=jnp.float32)


def _tc_bn_pool(y, ssum, ssq, g, be, batch2):
    return pl.pallas_call(
        _bn_pool_body,
        grid=(GRID,),
        in_specs=[
            pl.BlockSpec((R, D), lambda i: (i, 0)),
            pl.BlockSpec((1, D), lambda i: (0, 0)),
            pl.BlockSpec((1, D), lambda i: (0, 0)),
            pl.BlockSpec((1, D), lambda i: (0, 0)),
            pl.BlockSpec((1, D), lambda i: (0, 0)),
            pl.BlockSpec((R, 1), lambda i: (i, 0)),
        ],
        out_specs=[
            pl.BlockSpec((NB, D), lambda i: (0, 0)),
            pl.BlockSpec((NB, 1), lambda i: (0, 0)),
        ],
        out_shape=[
            jax.ShapeDtypeStruct((NB, D), jnp.float32),
            jax.ShapeDtypeStruct((NB, 1), jnp.float32),
        ],
    )(y, ssum, ssq, g, be, batch2)


def _head_body(segs_ref, cnt_ref, w1_ref, b1_ref, w2_ref, b2_ref, out_ref):
    pooled = segs_ref[...] / jnp.maximum(cnt_ref[...], 1.0)
    o = jnp.dot(pooled, w1_ref[...], preferred_element_type=jnp.float32)
    o = o + b1_ref[...]
    o = jnp.dot(o, w2_ref[...], preferred_element_type=jnp.float32)
    o = o + b2_ref[...]
    m = jnp.max(o, axis=-1, keepdims=True)
    lse = m + jnp.log(jnp.sum(jnp.exp(o - m), axis=-1, keepdims=True))
    out_ref[...] = o - lse


def _tc_head(segs, cnt, l1W, l1b, l2W, l2b):
    return pl.pallas_call(
        _head_body,
        out_shape=jax.ShapeDtypeStruct((NB, 4), jnp.float32),
    )(segs, cnt, l1W, l1b.reshape(1, -1), l2W, l2b.reshape(1, -1))


def kernel(x, edge_index, batch, W1, b1, g1, be1, W2, b2, g2, be2,
           l1W, l1b, l2W, l2b):
    src = edge_index[0]
    dst = edge_index[1]
    batch2 = batch.astype(jnp.int32).reshape(N, 1)
    zrows = jnp.zeros((ZR, D), jnp.float32)
    onesr = jnp.ones((K, D), jnp.float32)

    npad = EPAD - E
    srcf = jnp.concatenate([src, jnp.zeros((npad,), src.dtype)])
    dstf = jnp.concatenate(
        [dst, N + (jnp.arange(npad, dtype=dst.dtype) % SINK)])
    dst3 = dstf.reshape(NW, CH, K)

    degp = _sc_deg(dst3, onesr, zrows)

    Ws = [W1[0], W1[1], W1[2], W2[0], W2[1], W2[2]]
    bs = [b1[0].reshape(1, D), b1[1].reshape(1, D), b1[2].reshape(1, D),
          b2[0].reshape(1, D), b2[1].reshape(1, D), b2[2].reshape(1, D)]
    gs = [g1[0].reshape(1, D), g1[1].reshape(1, D), g1[2].reshape(1, D),
          g2[0].reshape(1, D), g2[1].reshape(1, D), g2[2].reshape(1, D)]
    bes = [be1[0].reshape(1, D), be1[1].reshape(1, D), be1[2].reshape(1, D),
           be2[0].reshape(1, D), be2[1].reshape(1, D), be2[2].reshape(1, D)]

    h, hp, dis = _tc_lead(x, Ws[0], bs[0], degp)
    segs = cnt = None
    for l in range(6):
        p = _sc_edge_agg(hp, srcf, dstf, zrows)
        y, ssum, ssq = _tc_relu_stats(p, h, dis)
        if l < 5:
            h, hp = _tc_bn_matmul(y, ssum, ssq, gs[l], bes[l],
                                  Ws[l + 1], bs[l + 1], dis)
        else:
            segs, cnt = _tc_bn_pool(y, ssum, ssq, gs[l], bes[l], batch2)
    return _tc_head(segs, cnt, l1W, l1b, l2W, l2b)


# speedup vs baseline: 1.0220x; 1.0220x over previous
import functools

import jax
import jax.numpy as jnp
from jax import lax
from jax.experimental import pallas as pl
from jax.experimental.pallas import tpu as pltpu
from jax.experimental.pallas import tpu_sc as plsc

N = 10000
E = 320000
D = 128
NB = 64

NC = 2
NS = 16
NW = NC * NS
K = 128
CH = 80
NPAIR = CH // 2
EPW = CH * K
EPAD = NW * EPW
SINK = 128
NA = N + SINK
RPS = 624
TAIL = N - NS * RPS
ZR = 208

R = 2000
GRID = N // R

DW = 16


@functools.cache
def _make_sc_edge_agg():
    mesh = plsc.VectorSubcoreMesh(core_axis_name="c", subcore_axis_name="s")

    @functools.partial(
        pl.kernel, mesh=mesh,
        out_type=jax.ShapeDtypeStruct((NC, N, D), jnp.float32),
        scratch_types=[
            pltpu.VMEM((2, K), jnp.int32),
            pltpu.VMEM((2, K), jnp.int32),
            pltpu.VMEM((K, D), jnp.float32),
            pltpu.VMEM((K, D), jnp.float32),
            pltpu.VMEM_SHARED((NA, D), jnp.float32),
            pltpu.SemaphoreType.DMA,
        ],
    )
    def sc_edge_agg(hp_hbm, sd_hbm, zrows_hbm, out_hbm,
                    sd_a, sd_b, rows_a, rows_b, acc, gsem):
        c = lax.axis_index("c")
        s = lax.axis_index("s")
        wid = c * NS + s
        for j in range(RPS // ZR):
            pltpu.sync_copy(zrows_hbm, acc.at[pl.ds(s * RPS + j * ZR, ZR)])

        @pl.when(s == 0)
        def _():
            pltpu.sync_copy(zrows_hbm.at[pl.ds(0, TAIL)],
                            acc.at[pl.ds(NS * RPS, TAIL)])
        plsc.subcore_barrier()

        pltpu.sync_copy(sd_hbm.at[wid, 0], sd_a)

        def pair(j, carry):
            i1 = 2 * j + 1
            g0 = pltpu.async_copy(hp_hbm.at[sd_a.at[0]], rows_a, gsem)

            @pl.when(j > 0)
            def _():
                pltpu.sync_copy(rows_b, acc.at[sd_b.at[1]], add=True)

            pltpu.sync_copy(sd_hbm.at[wid, i1], sd_b)
            g0.wait()
            g1 = pltpu.async_copy(hp_hbm.at[sd_b.at[0]], rows_b, gsem)
            pltpu.sync_copy(rows_a, acc.at[sd_a.at[1]], add=True)

            @pl.when(j < NPAIR - 1)
            def _():
                pltpu.sync_copy(sd_hbm.at[wid, i1 + 1], sd_a)

            g1.wait()
            return carry

        lax.fori_loop(0, NPAIR, pair, 0)
        pltpu.sync_copy(rows_b, acc.at[sd_b.at[1]], add=True)
        plsc.subcore_barrier()
        pltpu.sync_copy(acc.at[pl.ds(s * RPS, RPS)],
                        out_hbm.at[c, pl.ds(s * RPS, RPS)])

        @pl.when(s == 0)
        def _():
            pltpu.sync_copy(acc.at[pl.ds(NS * RPS, TAIL)],
                            out_hbm.at[c, pl.ds(NS * RPS, TAIL)])

    return sc_edge_agg


def _sc_edge_agg(hp, sd, zrows):
    return _make_sc_edge_agg()(hp, sd, zrows)


@functools.cache
def _make_sc_deg():
    mesh = plsc.VectorSubcoreMesh(core_axis_name="c", subcore_axis_name="s")

    @functools.partial(
        pl.kernel, mesh=mesh,
        out_type=jax.ShapeDtypeStruct((NC, N, D), jnp.float32),
        scratch_types=[
            pltpu.VMEM((CH, K), jnp.int32),
            pltpu.VMEM((K, D), jnp.float32),
            pltpu.VMEM_SHARED((NA, D), jnp.float32),
        ],
    )
    def sc_deg(dst3_hbm, ones_hbm, zrows_hbm, out_hbm, dsts_v, ones_v, acc):
        c = lax.axis_index("c")
        s = lax.axis_index("s")
        wid = c * NS + s
        pltpu.sync_copy(dst3_hbm.at[wid], dsts_v)
        for j in range(RPS // ZR):
            pltpu.sync_copy(zrows_hbm, acc.at[pl.ds(s * RPS + j * ZR, ZR)])

        @pl.when(s == 0)
        def _():
            pltpu.sync_copy(zrows_hbm.at[pl.ds(0, TAIL)],
                            acc.at[pl.ds(NS * RPS, TAIL)])
        pltpu.sync_copy(ones_hbm, ones_v)
        plsc.subcore_barrier()

        def step(i, carry):
            pltpu.sync_copy(ones_v, acc.at[dsts_v.at[i]], add=True)
            return carry

        lax.fori_loop(0, CH, step, 0)
        plsc.subcore_barrier()
        pltpu.sync_copy(acc.at[pl.ds(s * RPS, RPS)],
                        out_hbm.at[c, pl.ds(s * RPS, RPS)])

        @pl.when(s == 0)
        def _():
            pltpu.sync_copy(acc.at[pl.ds(NS * RPS, TAIL)],
                            out_hbm.at[c, pl.ds(NS * RPS, TAIL)])

    return sc_deg


def _sc_deg(dst3, onesr, zrows):
    return _make_sc_deg()(dst3, onesr, zrows)


def _lead_body(x_ref, w_ref, b_ref, degp_ref, h_ref, hp_ref, dis_ref):
    deg = degp_ref[0, :, 0:1] + degp_ref[1, :, 0:1] + 1.0
    dis = lax.rsqrt(deg)
    h = jnp.dot(x_ref[...], w_ref[...], preferred_element_type=jnp.float32)
    h = h + b_ref[...]
    h_ref[...] = h
    hp_ref[...] = h * dis
    dis_ref[...] = dis


def _tc_lead(x, w, b, degp):
    return pl.pallas_call(
        _lead_body,
        grid=(GRID,),
        in_specs=[
            pl.BlockSpec((R, D), lambda i: (i, 0)),
            pl.BlockSpec((D, D), lambda i: (0, 0)),
            pl.BlockSpec((1, D), lambda i: (0, 0)),
            pl.BlockSpec((NC, R, D), lambda i: (0, i, 0)),
        ],
        out_specs=[
            pl.BlockSpec((R, D), lambda i: (i, 0)),
            pl.BlockSpec((R, D), lambda i: (i, 0)),
            pl.BlockSpec((R, 1), lambda i: (i, 0)),
        ],
        out_shape=[
            jax.ShapeDtypeStruct((N, D), jnp.float32),
            jax.ShapeDtypeStruct((N, D), jnp.float32),
            jax.ShapeDtypeStruct((N, 1), jnp.float32),
        ],
    )(x, w, b, degp)


def _relu_stats_body(p_ref, h_ref, dis_ref, y_ref, ssum_ref, ssq_ref):
    dis = dis_ref[...]
    agg = dis * (p_ref[0] + p_ref[1]) + (dis * dis) * h_ref[...]
    y = jnp.maximum(agg, 0.0)
    y_ref[...] = y

    @pl.when(pl.program_id(0) == 0)
    def _():
        ssum_ref[...] = jnp.zeros_like(ssum_ref)
        ssq_ref[...] = jnp.zeros_like(ssq_ref)

    ssum_ref[...] += jnp.sum(y, axis=0, keepdims=True)
    ssq_ref[...] += jnp.sum(y * y, axis=0, keepdims=True)


def _tc_relu_stats(p, h, dis):
    return pl.pallas_call(
        _relu_stats_body,
        grid=(GRID,),
        in_specs=[
            pl.BlockSpec((NC, R, D), lambda i: (0, i, 0)),
            pl.BlockSpec((R, D), lambda i: (i, 0)),
            pl.BlockSpec((R, 1), lambda i: (i, 0)),
        ],
        out_specs=[
            pl.BlockSpec((R, D), lambda i: (i, 0)),
            pl.BlockSpec((1, D), lambda i: (0, 0)),
            pl.BlockSpec((1, D), lambda i: (0, 0)),
        ],
        out_shape=[
            jax.ShapeDtypeStruct((N, D), jnp.float32),
            jax.ShapeDtypeStruct((1, D), jnp.float32),
            jax.ShapeDtypeStruct((1, D), jnp.float32),
        ],
    )(p, h, dis)


def _bn_matmul_body(y_ref, ssum_ref, ssq_ref, g_ref, be_ref, w_ref, b_ref,
                    dis_ref, h_ref, hp_ref):
    mu = ssum_ref[...] / N
    var = ssq_ref[...] / N - mu * mu
    rstd = lax.rsqrt(var + 1e-5)
    xn = (y_ref[...] - mu) * (rstd * g_ref[...]) + be_ref[...]
    h = jnp.dot(xn, w_ref[...], preferred_element_type=jnp.float32)
    h = h + b_ref[...]
    h_ref[...] = h
    hp_ref[...] = h * dis_ref[...]


def _tc_bn_matmul(y, ssum, ssq, g, be, w, b, dis):
    return pl.pallas_call(
        _bn_matmul_body,
        grid=(GRID,),
        in_specs=[
            pl.BlockSpec((R, D), lambda i: (i, 0)),
            pl.BlockSpec((1, D), lambda i: (0, 0)),
            pl.BlockSpec((1, D), lambda i: (0, 0)),
            pl.BlockSpec((1, D), lambda i: (0, 0)),
            pl.BlockSpec((1, D), lambda i: (0, 0)),
            pl.BlockSpec((D, D), lambda i: (0, 0)),
            pl.BlockSpec((1, D), lambda i: (0, 0)),
            pl.BlockSpec((R, 1), lambda i: (i, 0)),
        ],
        out_specs=[
            pl.BlockSpec((R, D), lambda i: (i, 0)),
            pl.BlockSpec((R, D), lambda i: (i, 0)),
        ],
        out_shape=[
            jax.ShapeDtypeStruct((N, D), jnp.float32),
            jax.ShapeDtypeStruct((N, D), jnp.float32),
        ],
    )(y, ssum, ssq, g, be, w, b, dis)


def _bn_pool_body(y_ref, ssum_ref, ssq_ref, g_ref, be_ref, batch_ref,
                  segs_ref, cnt_ref):
    mu = ssum_ref[...] / N
    var = ssq_ref[...] / N - mu * mu
    rstd = lax.rsqrt(var + 1e-5)
    xn = (y_ref[...] - mu) * (rstd * g_ref[...]) + be_ref[...]
    seg_ids = lax.broadcasted_iota(jnp.int32, (1, NB), 1)
    oneh = jnp.where(batch_ref[...] == seg_ids, 1.0, 0.0)

    @pl.when(pl.program_id(0) == 0)
    def _():
        segs_ref[...] = jnp.zeros_like(segs_ref)
        cnt_ref[...] = jnp.zeros_like(cnt_ref)

    segs_ref[...] += lax.dot_general(oneh, xn, (((0,), (0,)), ((), ())),
                                     preferred_element_type=jnp.float32)
    ones_col = jnp.ones((R, 1), jnp.float32)
    cnt_ref[...] += lax.dot_general(oneh, ones_col, (((0,), (0,)), ((), ())),
                                    preferred_element_type=jnp.float32)


def _tc_bn_pool(y, ssum, ssq, g, be, batch2):
    return pl.pallas_call(
        _bn_pool_body,
        grid=(GRID,),
        in_specs=[
            pl.BlockSpec((R, D), lambda i: (i, 0)),
            pl.BlockSpec((1, D), lambda i: (0, 0)),
            pl.BlockSpec((1, D), lambda i: (0, 0)),
            pl.BlockSpec((1, D), lambda i: (0, 0)),
            pl.BlockSpec((1, D), lambda i: (0, 0)),
            pl.BlockSpec((R, 1), lambda i: (i, 0)),
        ],
        out_specs=[
            pl.BlockSpec((NB, D), lambda i: (0, 0)),
            pl.BlockSpec((NB, 1), lambda i: (0, 0)),
        ],
        out_shape=[
            jax.ShapeDtypeStruct((NB, D), jnp.float32),
            jax.ShapeDtypeStruct((NB, 1), jnp.float32),
        ],
    )(y, ssum, ssq, g, be, batch2)


def _head_body(segs_ref, cnt_ref, w1_ref, b1_ref, w2_ref, b2_ref, out_ref):
    pooled = segs_ref[...] / jnp.maximum(cnt_ref[...], 1.0)
    o = jnp.dot(pooled, w1_ref[...], preferred_element_type=jnp.float32)
    o = o + b1_ref[...]
    o = jnp.dot(o, w2_ref[...], preferred_element_type=jnp.float32)
    o = o + b2_ref[...]
    m = jnp.max(o, axis=-1, keepdims=True)
    lse = m + jnp.log(jnp.sum(jnp.exp(o - m), axis=-1, keepdims=True))
    out_ref[...] = o - lse


def _tc_head(segs, cnt, l1W, l1b, l2W, l2b):
    return pl.pallas_call(
        _head_body,
        out_shape=jax.ShapeDtypeStruct((NB, 4), jnp.float32),
    )(segs, cnt, l1W, l1b.reshape(1, -1), l2W, l2b.reshape(1, -1))


def kernel(x, edge_index, batch, W1, b1, g1, be1, W2, b2, g2, be2,
           l1W, l1b, l2W, l2b):
    src = edge_index[0]
    dst = edge_index[1]
    batch2 = batch.astype(jnp.int32).reshape(N, 1)
    zrows = jnp.zeros((ZR, D), jnp.float32)
    onesr = jnp.ones((K, D), jnp.float32)

    npad = EPAD - E
    srcf = jnp.concatenate([src, jnp.zeros((npad,), src.dtype)])
    dstf = jnp.concatenate(
        [dst, N + (jnp.arange(npad, dtype=dst.dtype) % SINK)])
    dst3 = dstf.reshape(NW, CH, K)
    sd = jnp.stack([srcf.reshape(NW, CH, K), dst3], axis=2)

    degp = _sc_deg(dst3, onesr, zrows)

    Ws = [W1[0], W1[1], W1[2], W2[0], W2[1], W2[2]]
    bs = [b1[0].reshape(1, D), b1[1].reshape(1, D), b1[2].reshape(1, D),
          b2[0].reshape(1, D), b2[1].reshape(1, D), b2[2].reshape(1, D)]
    gs = [g1[0].reshape(1, D), g1[1].reshape(1, D), g1[2].reshape(1, D),
          g2[0].reshape(1, D), g2[1].reshape(1, D), g2[2].reshape(1, D)]
    bes = [be1[0].reshape(1, D), be1[1].reshape(1, D), be1[2].reshape(1, D),
           be2[0].reshape(1, D), be2[1].reshape(1, D), be2[2].reshape(1, D)]

    h, hp, dis = _tc_lead(x, Ws[0], bs[0], degp)
    segs = cnt = None
    for l in range(6):
        p = _sc_edge_agg(hp, sd, zrows)
        y, ssum, ssq = _tc_relu_stats(p, h, dis)
        if l < 5:
            h, hp = _tc_bn_matmul(y, ssum, ssq, gs[l], bes[l],
                                  Ws[l + 1], bs[l + 1], dis)
        else:
            segs, cnt = _tc_bn_pool(y, ssum, ssq, gs[l], bes[l], batch2)
    return _tc_head(segs, cnt, l1W, l1b, l2W, l2b)


# speedup vs baseline: 2.9830x; 2.9187x over previous
import functools

import jax
import jax.numpy as jnp
from jax import lax
from jax.experimental import pallas as pl
from jax.experimental.pallas import tpu as pltpu
from jax.experimental.pallas import tpu_sc as plsc

N = 10000
E = 320000
D = 128
NB = 64

NC = 2
NS = 16
NW = NC * NS
K = 128
CH = 80
NPAIR = CH // 2
EPW = CH * K
EPAD = NW * EPW
SINK = 128
NA = N + SINK
RPS = 624
TAIL = N - NS * RPS
ZR = 208

R = 2000
GRID = N // R

DW = 16


@functools.cache
def _make_sc_edge_agg():
    mesh = plsc.VectorSubcoreMesh(core_axis_name="c", subcore_axis_name="s")

    @functools.partial(
        pl.kernel, mesh=mesh,
        out_type=jax.ShapeDtypeStruct((NC, N, D), jnp.float32),
        scratch_types=[
            pltpu.VMEM((2, K), jnp.int32),
            pltpu.VMEM((2, K), jnp.int32),
            pltpu.VMEM((K, D), jnp.float32),
            pltpu.VMEM((K, D), jnp.float32),
            pltpu.VMEM_SHARED((NA, D), jnp.float32),
            pltpu.SemaphoreType.DMA,
        ],
    )
    def sc_edge_agg(hp_hbm, sd_hbm, zrows_hbm, out_hbm,
                    sd_a, sd_b, rows_a, rows_b, acc, gsem):
        c = lax.axis_index("c")
        s = lax.axis_index("s")
        wid = c * NS + s
        for j in range(RPS // ZR):
            pltpu.sync_copy(zrows_hbm, acc.at[pl.ds(s * RPS + j * ZR, ZR)])

        @pl.when(s == 0)
        def _():
            pltpu.sync_copy(zrows_hbm.at[pl.ds(0, TAIL)],
                            acc.at[pl.ds(NS * RPS, TAIL)])
        plsc.subcore_barrier()

        pltpu.sync_copy(sd_hbm.at[wid, 0], sd_a)

        def pair(j, carry):
            i1 = 2 * j + 1
            g0 = pltpu.async_copy(hp_hbm.at[sd_a.at[0]], rows_a, gsem)

            @pl.when(j > 0)
            def _():
                pltpu.sync_copy(rows_b, acc.at[sd_b.at[1]], add=True)

            pltpu.sync_copy(sd_hbm.at[wid, i1], sd_b)
            g0.wait()
            g1 = pltpu.async_copy(hp_hbm.at[sd_b.at[0]], rows_b, gsem)
            pltpu.sync_copy(rows_a, acc.at[sd_a.at[1]], add=True)

            @pl.when(j < NPAIR - 1)
            def _():
                pltpu.sync_copy(sd_hbm.at[wid, i1 + 1], sd_a)

            g1.wait()
            return carry

        lax.fori_loop(0, NPAIR, pair, 0)
        pltpu.sync_copy(rows_b, acc.at[sd_b.at[1]], add=True)
        plsc.subcore_barrier()
        pltpu.sync_copy(acc.at[pl.ds(s * RPS, RPS)],
                        out_hbm.at[c, pl.ds(s * RPS, RPS)])

        @pl.when(s == 0)
        def _():
            pltpu.sync_copy(acc.at[pl.ds(NS * RPS, TAIL)],
                            out_hbm.at[c, pl.ds(NS * RPS, TAIL)])

    return sc_edge_agg


def _sc_edge_agg(hp, sd, zrows):
    return _make_sc_edge_agg()(hp, sd, zrows)


@functools.cache
def _make_sc_deg():
    mesh = plsc.VectorSubcoreMesh(core_axis_name="c", subcore_axis_name="s")

    @functools.partial(
        pl.kernel, mesh=mesh,
        out_type=jax.ShapeDtypeStruct((NC, N, D), jnp.float32),
        scratch_types=[
            pltpu.VMEM((CH, K), jnp.int32),
            pltpu.VMEM((K, D), jnp.float32),
            pltpu.VMEM_SHARED((NA, D), jnp.float32),
        ],
    )
    def sc_deg(dst3_hbm, ones_hbm, zrows_hbm, out_hbm, dsts_v, ones_v, acc):
        c = lax.axis_index("c")
        s = lax.axis_index("s")
        wid = c * NS + s
        pltpu.sync_copy(dst3_hbm.at[wid], dsts_v)
        for j in range(RPS // ZR):
            pltpu.sync_copy(zrows_hbm, acc.at[pl.ds(s * RPS + j * ZR, ZR)])

        @pl.when(s == 0)
        def _():
            pltpu.sync_copy(zrows_hbm.at[pl.ds(0, TAIL)],
                            acc.at[pl.ds(NS * RPS, TAIL)])
        pltpu.sync_copy(ones_hbm, ones_v)
        plsc.subcore_barrier()

        def step(i, carry):
            pltpu.sync_copy(ones_v, acc.at[dsts_v.at[i]], add=True)
            return carry

        lax.fori_loop(0, CH, step, 0)
        plsc.subcore_barrier()
        pltpu.sync_copy(acc.at[pl.ds(s * RPS, RPS)],
                        out_hbm.at[c, pl.ds(s * RPS, RPS)])

        @pl.when(s == 0)
        def _():
            pltpu.sync_copy(acc.at[pl.ds(NS * RPS, TAIL)],
                            out_hbm.at[c, pl.ds(NS * RPS, TAIL)])

    return sc_deg


def _sc_deg(dst3, onesr, zrows):
    return _make_sc_deg()(dst3, onesr, zrows)


def _lead_body(x_ref, w_ref, b_ref, degp_ref, h_ref, hp_ref, dis_ref):
    deg = degp_ref[0, :, 0:1] + degp_ref[1, :, 0:1] + 1.0
    dis = lax.rsqrt(deg)
    h = jnp.dot(x_ref[...], w_ref[...], preferred_element_type=jnp.float32)
    h = h + b_ref[...]
    h_ref[...] = h
    hp_ref[...] = h * dis
    dis_ref[...] = dis


def _tc_lead(x, w, b, degp):
    return pl.pallas_call(
        _lead_body,
        grid=(GRID,),
        in_specs=[
            pl.BlockSpec((R, D), lambda i: (i, 0)),
            pl.BlockSpec((D, D), lambda i: (0, 0)),
            pl.BlockSpec((1, D), lambda i: (0, 0)),
            pl.BlockSpec((NC, R, D), lambda i: (0, i, 0)),
        ],
        out_specs=[
            pl.BlockSpec((R, D), lambda i: (i, 0)),
            pl.BlockSpec((R, D), lambda i: (i, 0)),
            pl.BlockSpec((R, 1), lambda i: (i, 0)),
        ],
        out_shape=[
            jax.ShapeDtypeStruct((N, D), jnp.float32),
            jax.ShapeDtypeStruct((N, D), jnp.float32),
            jax.ShapeDtypeStruct((N, 1), jnp.float32),
        ],
    )(x, w, b, degp)


def _relu_stats_body(p_ref, h_ref, dis_ref, y_ref, ssum_ref, ssq_ref):
    dis = dis_ref[...]
    agg = dis * (p_ref[0] + p_ref[1]) + (dis * dis) * h_ref[...]
    y = jnp.maximum(agg, 0.0)
    y_ref[...] = y

    @pl.when(pl.program_id(0) == 0)
    def _():
        ssum_ref[...] = jnp.zeros_like(ssum_ref)
        ssq_ref[...] = jnp.zeros_like(ssq_ref)

    ssum_ref[...] += jnp.sum(y, axis=0, keepdims=True)
    ssq_ref[...] += jnp.sum(y * y, axis=0, keepdims=True)


def _tc_relu_stats(p, h, dis):
    return pl.pallas_call(
        _relu_stats_body,
        grid=(GRID,),
        in_specs=[
            pl.BlockSpec((NC, R, D), lambda i: (0, i, 0)),
            pl.BlockSpec((R, D), lambda i: (i, 0)),
            pl.BlockSpec((R, 1), lambda i: (i, 0)),
        ],
        out_specs=[
            pl.BlockSpec((R, D), lambda i: (i, 0)),
            pl.BlockSpec((1, D), lambda i: (0, 0)),
            pl.BlockSpec((1, D), lambda i: (0, 0)),
        ],
        out_shape=[
            jax.ShapeDtypeStruct((N, D), jnp.float32),
            jax.ShapeDtypeStruct((1, D), jnp.float32),
            jax.ShapeDtypeStruct((1, D), jnp.float32),
        ],
    )(p, h, dis)


def _bn_matmul_body(y_ref, ssum_ref, ssq_ref, g_ref, be_ref, w_ref, b_ref,
                    dis_ref, h_ref, hp_ref):
    mu = ssum_ref[...] / N
    var = ssq_ref[...] / N - mu * mu
    rstd = lax.rsqrt(var + 1e-5)
    xn = (y_ref[...] - mu) * (rstd * g_ref[...]) + be_ref[...]
    h = jnp.dot(xn, w_ref[...], preferred_element_type=jnp.float32)
    h = h + b_ref[...]
    h_ref[...] = h
    hp_ref[...] = h * dis_ref[...]


def _tc_bn_matmul(y, ssum, ssq, g, be, w, b, dis):
    return pl.pallas_call(
        _bn_matmul_body,
        grid=(GRID,),
        in_specs=[
            pl.BlockSpec((R, D), lambda i: (i, 0)),
            pl.BlockSpec((1, D), lambda i: (0, 0)),
            pl.BlockSpec((1, D), lambda i: (0, 0)),
            pl.BlockSpec((1, D), lambda i: (0, 0)),
            pl.BlockSpec((1, D), lambda i: (0, 0)),
            pl.BlockSpec((D, D), lambda i: (0, 0)),
            pl.BlockSpec((1, D), lambda i: (0, 0)),
            pl.BlockSpec((R, 1), lambda i: (i, 0)),
        ],
        out_specs=[
            pl.BlockSpec((R, D), lambda i: (i, 0)),
            pl.BlockSpec((R, D), lambda i: (i, 0)),
        ],
        out_shape=[
            jax.ShapeDtypeStruct((N, D), jnp.float32),
            jax.ShapeDtypeStruct((N, D), jnp.float32),
        ],
    )(y, ssum, ssq, g, be, w, b, dis)


def _bn_pool_body(y_ref, ssum_ref, ssq_ref, g_ref, be_ref, batch_ref,
                  segs_ref, cnt_ref):
    mu = ssum_ref[...] / N
    var = ssq_ref[...] / N - mu * mu
    rstd = lax.rsqrt(var + 1e-5)
    xn = (y_ref[...] - mu) * (rstd * g_ref[...]) + be_ref[...]
    seg_ids = lax.broadcasted_iota(jnp.int32, (1, NB), 1)
    oneh = jnp.where(batch_ref[...] == seg_ids, 1.0, 0.0)

    @pl.when(pl.program_id(0) == 0)
    def _():
        segs_ref[...] = jnp.zeros_like(segs_ref)
        cnt_ref[...] = jnp.zeros_like(cnt_ref)

    segs_ref[...] += lax.dot_general(oneh, xn, (((0,), (0,)), ((), ())),
                                     preferred_element_type=jnp.float32)
    ones_col = jnp.ones((R, 1), jnp.float32)
    cnt_ref[...] += lax.dot_general(oneh, ones_col, (((0,), (0,)), ((), ())),
                                    preferred_element_type=jnp.float32)


def _tc_bn_pool(y, ssum, ssq, g, be, batch2):
    return pl.pallas_call(
        _bn_pool_body,
        grid=(GRID,),
        in_specs=[
            pl.BlockSpec((R, D), lambda i: (i, 0)),
            pl.BlockSpec((1, D), lambda i: (0, 0)),
            pl.BlockSpec((1, D), lambda i: (0, 0)),
            pl.BlockSpec((1, D), lambda i: (0, 0)),
            pl.BlockSpec((1, D), lambda i: (0, 0)),
            pl.BlockSpec((R, 1), lambda i: (i, 0)),
        ],
        out_specs=[
            pl.BlockSpec((NB, D), lambda i: (0, 0)),
            pl.BlockSpec((NB, 1), lambda i: (0, 0)),
        ],
        out_shape=[
            jax.ShapeDtypeStruct((NB, D), jnp.float32),
            jax.ShapeDtypeStruct((NB, 1), jnp.float32),
        ],
    )(y, ssum, ssq, g, be, batch2)


def _head_body(segs_ref, cnt_ref, w1_ref, b1_ref, w2_ref, b2_ref, out_ref):
    pooled = segs_ref[...] / jnp.maximum(cnt_ref[...], 1.0)
    o = jnp.dot(pooled, w1_ref[...], preferred_element_type=jnp.float32)
    o = o + b1_ref[...]
    o = jnp.dot(o, w2_ref[...], preferred_element_type=jnp.float32)
    o = o + b2_ref[...]
    m = jnp.max(o, axis=-1, keepdims=True)
    lse = m + jnp.log(jnp.sum(jnp.exp(o - m), axis=-1, keepdims=True))
    out_ref[...] = o - lse


def _tc_head(segs, cnt, l1W, l1b, l2W, l2b):
    return pl.pallas_call(
        _head_body,
        out_shape=jax.ShapeDtypeStruct((NB, 4), jnp.float32),
    )(segs, cnt, l1W, l1b.reshape(1, -1), l2W, l2b.reshape(1, -1))


def kernel(x, edge_index, batch, W1, b1, g1, be1, W2, b2, g2, be2,
           l1W, l1b, l2W, l2b):
    src = edge_index[0]
    dst = edge_index[1]
    batch2 = batch.astype(jnp.int32).reshape(N, 1)
    zrows = jnp.zeros((ZR, D), jnp.float32)
    onesr = jnp.ones((K, D), jnp.float32)

    ppw = EPW - E // NW
    pad_ar = jnp.arange(ppw, dtype=src.dtype)
    src_pad = jnp.tile(pad_ar[None, :], (NW, 1))
    dst_pad = jnp.tile(N + (pad_ar % SINK)[None, :], (NW, 1))
    src3 = jnp.concatenate(
        [src.reshape(NW, -1), src_pad], axis=1).reshape(NW, CH, K)
    dst3 = jnp.concatenate(
        [dst.reshape(NW, -1), dst_pad], axis=1).reshape(NW, CH, K)
    sd = jnp.stack([src3, dst3], axis=2)

    degp = _sc_deg(dst3, onesr, zrows)

    Ws = [W1[0], W1[1], W1[2], W2[0], W2[1], W2[2]]
    bs = [b1[0].reshape(1, D), b1[1].reshape(1, D), b1[2].reshape(1, D),
          b2[0].reshape(1, D), b2[1].reshape(1, D), b2[2].reshape(1, D)]
    gs = [g1[0].reshape(1, D), g1[1].reshape(1, D), g1[2].reshape(1, D),
          g2[0].reshape(1, D), g2[1].reshape(1, D), g2[2].reshape(1, D)]
    bes = [be1[0].reshape(1, D), be1[1].reshape(1, D), be1[2].reshape(1, D),
           be2[0].reshape(1, D), be2[1].reshape(1, D), be2[2].reshape(1, D)]

    h, hp, dis = _tc_lead(x, Ws[0], bs[0], degp)
    segs = cnt = None
    for l in range(6):
        p = _sc_edge_agg(hp, sd, zrows)
        y, ssum, ssq = _tc_relu_stats(p, h, dis)
        if l < 5:
            h, hp = _tc_bn_matmul(y, ssum, ssq, gs[l], bes[l],
                                  Ws[l + 1], bs[l + 1], dis)
        else:
            segs, cnt = _tc_bn_pool(y, ssum, ssq, gs[l], bes[l], batch2)
    return _tc_head(segs, cnt, l1W, l1b, l2W, l2b)


# speedup vs baseline: 3.5118x; 1.1773x over previous
import functools

import jax
import jax.numpy as jnp
from jax import lax
from jax.experimental import pallas as pl
from jax.experimental.pallas import tpu as pltpu
from jax.experimental.pallas import tpu_sc as plsc

N = 10000
E = 320000
D = 128
NB = 64

NC = 2
NS = 16
NW = NC * NS
K = 128
CH = 81
NTRI = CH // 3
EPW = CH * K
EPAD = NW * EPW
SINK = 64
NA = N + SINK
RPS = 624
TAIL = N - NS * RPS
ZR = 208

R = 2000
GRID = N // R

DW = 16


@functools.cache
def _make_sc_edge_agg():
    mesh = plsc.VectorSubcoreMesh(core_axis_name="c", subcore_axis_name="s")

    @functools.partial(
        pl.kernel, mesh=mesh,
        out_type=jax.ShapeDtypeStruct((NC, N, D), jnp.float32),
        scratch_types=[
            pltpu.VMEM((2, K), jnp.int32),
            pltpu.VMEM((2, K), jnp.int32),
            pltpu.VMEM((2, K), jnp.int32),
            pltpu.VMEM((K, D), jnp.float32),
            pltpu.VMEM((K, D), jnp.float32),
            pltpu.VMEM((K, D), jnp.float32),
            pltpu.VMEM_SHARED((NA, D), jnp.float32),
            pltpu.SemaphoreType.DMA,
            pltpu.SemaphoreType.DMA,
        ],
    )
    def sc_edge_agg(hp_hbm, sd_hbm, zrows_hbm, out_hbm,
                    sd0, sd1, sd2, r0, r1, r2, acc, gsem, ssem):
        c = lax.axis_index("c")
        s = lax.axis_index("s")
        wid = c * NS + s
        for j in range(RPS // ZR):
            pltpu.sync_copy(zrows_hbm, acc.at[pl.ds(s * RPS + j * ZR, ZR)])

        @pl.when(s == 0)
        def _():
            pltpu.sync_copy(zrows_hbm.at[pl.ds(0, TAIL)],
                            acc.at[pl.ds(NS * RPS, TAIL)])
        plsc.subcore_barrier()

        bufs = ((sd0, r0), (sd1, r1), (sd2, r2))
        pltpu.sync_copy(sd_hbm.at[wid, 0], sd0)
        pltpu.async_copy(hp_hbm.at[sd0.at[0]], r0, gsem)
        pltpu.sync_copy(sd_hbm.at[wid, 1], sd1)
        pltpu.async_copy(hp_hbm.at[sd1.at[0]], r1, gsem)

        def tri(j, carry):
            for t in range(3):
                i = 3 * j + t
                sdX, rX = bufs[t]
                sdY, rY = bufs[(t + 2) % 3]
                pltpu.make_async_copy(hp_hbm.at[sdX.at[0]], rX, gsem).wait()
                pltpu.async_copy(rX, acc.at[sdX.at[1]], ssem, add=True)

                def drain_prev():
                    pltpu.make_async_copy(
                        rY, acc.at[sdY.at[1]], ssem).wait()

                def fetch_next():
                    pltpu.sync_copy(sd_hbm.at[wid, i + 2], sdY)
                    pltpu.async_copy(hp_hbm.at[sdY.at[0]], rY, gsem)

                if t == 0:
                    pl.when(j > 0)(drain_prev)
                    fetch_next()
                else:
                    drain_prev()
                    pl.when(j < NTRI - 1)(fetch_next)
            return carry

        lax.fori_loop(0, NTRI, tri, 0)
        pltpu.make_async_copy(r2, acc.at[sd2.at[1]], ssem).wait()
        plsc.subcore_barrier()
        pltpu.sync_copy(acc.at[pl.ds(s * RPS, RPS)],
                        out_hbm.at[c, pl.ds(s * RPS, RPS)])

        @pl.when(s == 0)
        def _():
            pltpu.sync_copy(acc.at[pl.ds(NS * RPS, TAIL)],
                            out_hbm.at[c, pl.ds(NS * RPS, TAIL)])

    return sc_edge_agg


def _sc_edge_agg(hp, sd, zrows):
    return _make_sc_edge_agg()(hp, sd, zrows)


@functools.cache
def _make_sc_deg():
    mesh = plsc.VectorSubcoreMesh(core_axis_name="c", subcore_axis_name="s")

    @functools.partial(
        pl.kernel, mesh=mesh,
        out_type=jax.ShapeDtypeStruct((NC, N, D), jnp.float32),
        scratch_types=[
            pltpu.VMEM((CH, K), jnp.int32),
            pltpu.VMEM((K, D), jnp.float32),
            pltpu.VMEM_SHARED((NA, D), jnp.float32),
        ],
    )
    def sc_deg(dst3_hbm, ones_hbm, zrows_hbm, out_hbm, dsts_v, ones_v, acc):
        c = lax.axis_index("c")
        s = lax.axis_index("s")
        wid = c * NS + s
        pltpu.sync_copy(dst3_hbm.at[wid], dsts_v)
        for j in range(RPS // ZR):
            pltpu.sync_copy(zrows_hbm, acc.at[pl.ds(s * RPS + j * ZR, ZR)])

        @pl.when(s == 0)
        def _():
            pltpu.sync_copy(zrows_hbm.at[pl.ds(0, TAIL)],
                            acc.at[pl.ds(NS * RPS, TAIL)])
        pltpu.sync_copy(ones_hbm, ones_v)
        plsc.subcore_barrier()

        def step(i, carry):
            pltpu.sync_copy(ones_v, acc.at[dsts_v.at[i]], add=True)
            return carry

        lax.fori_loop(0, CH, step, 0)
        plsc.subcore_barrier()
        pltpu.sync_copy(acc.at[pl.ds(s * RPS, RPS)],
                        out_hbm.at[c, pl.ds(s * RPS, RPS)])

        @pl.when(s == 0)
        def _():
            pltpu.sync_copy(acc.at[pl.ds(NS * RPS, TAIL)],
                            out_hbm.at[c, pl.ds(NS * RPS, TAIL)])

    return sc_deg


def _sc_deg(dst3, onesr, zrows):
    return _make_sc_deg()(dst3, onesr, zrows)


def _lead_body(x_ref, w_ref, b_ref, degp_ref, h_ref, hp_ref, dis_ref):
    deg = degp_ref[0, :, 0:1] + degp_ref[1, :, 0:1] + 1.0
    dis = lax.rsqrt(deg)
    h = jnp.dot(x_ref[...], w_ref[...], preferred_element_type=jnp.float32)
    h = h + b_ref[...]
    h_ref[...] = h
    hp_ref[...] = h * dis
    dis_ref[...] = dis


def _tc_lead(x, w, b, degp):
    return pl.pallas_call(
        _lead_body,
        grid=(GRID,),
        in_specs=[
            pl.BlockSpec((R, D), lambda i: (i, 0)),
            pl.BlockSpec((D, D), lambda i: (0, 0)),
            pl.BlockSpec((1, D), lambda i: (0, 0)),
            pl.BlockSpec((NC, R, D), lambda i: (0, i, 0)),
        ],
        out_specs=[
            pl.BlockSpec((R, D), lambda i: (i, 0)),
            pl.BlockSpec((R, D), lambda i: (i, 0)),
            pl.BlockSpec((R, 1), lambda i: (i, 0)),
        ],
        out_shape=[
            jax.ShapeDtypeStruct((N, D), jnp.float32),
            jax.ShapeDtypeStruct((N, D), jnp.float32),
            jax.ShapeDtypeStruct((N, 1), jnp.float32),
        ],
    )(x, w, b, degp)


def _relu_stats_body(p_ref, h_ref, dis_ref, y_ref, ssum_ref, ssq_ref):
    dis = dis_ref[...]
    agg = dis * (p_ref[0] + p_ref[1]) + (dis * dis) * h_ref[...]
    y = jnp.maximum(agg, 0.0)
    y_ref[...] = y

    @pl.when(pl.program_id(0) == 0)
    def _():
        ssum_ref[...] = jnp.zeros_like(ssum_ref)
        ssq_ref[...] = jnp.zeros_like(ssq_ref)

    ssum_ref[...] += jnp.sum(y, axis=0, keepdims=True)
    ssq_ref[...] += jnp.sum(y * y, axis=0, keepdims=True)


def _tc_relu_stats(p, h, dis):
    return pl.pallas_call(
        _relu_stats_body,
        grid=(GRID,),
        in_specs=[
            pl.BlockSpec((NC, R, D), lambda i: (0, i, 0)),
            pl.BlockSpec((R, D), lambda i: (i, 0)),
            pl.BlockSpec((R, 1), lambda i: (i, 0)),
        ],
        out_specs=[
            pl.BlockSpec((R, D), lambda i: (i, 0)),
            pl.BlockSpec((1, D), lambda i: (0, 0)),
            pl.BlockSpec((1, D), lambda i: (0, 0)),
        ],
        out_shape=[
            jax.ShapeDtypeStruct((N, D), jnp.float32),
            jax.ShapeDtypeStruct((1, D), jnp.float32),
            jax.ShapeDtypeStruct((1, D), jnp.float32),
        ],
    )(p, h, dis)


def _bn_matmul_body(y_ref, ssum_ref, ssq_ref, g_ref, be_ref, w_ref, b_ref,
                    dis_ref, h_ref, hp_ref):
    mu = ssum_ref[...] / N
    var = ssq_ref[...] / N - mu * mu
    rstd = lax.rsqrt(var + 1e-5)
    xn = (y_ref[...] - mu) * (rstd * g_ref[...]) + be_ref[...]
    h = jnp.dot(xn, w_ref[...], preferred_element_type=jnp.float32)
    h = h + b_ref[...]
    h_ref[...] = h
    hp_ref[...] = h * dis_ref[...]


def _tc_bn_matmul(y, ssum, ssq, g, be, w, b, dis):
    return pl.pallas_call(
        _bn_matmul_body,
        grid=(GRID,),
        in_specs=[
            pl.BlockSpec((R, D), lambda i: (i, 0)),
            pl.BlockSpec((1, D), lambda i: (0, 0)),
            pl.BlockSpec((1, D), lambda i: (0, 0)),
            pl.BlockSpec((1, D), lambda i: (0, 0)),
            pl.BlockSpec((1, D), lambda i: (0, 0)),
            pl.BlockSpec((D, D), lambda i: (0, 0)),
            pl.BlockSpec((1, D), lambda i: (0, 0)),
            pl.BlockSpec((R, 1), lambda i: (i, 0)),
        ],
        out_specs=[
            pl.BlockSpec((R, D), lambda i: (i, 0)),
            pl.BlockSpec((R, D), lambda i: (i, 0)),
        ],
        out_shape=[
            jax.ShapeDtypeStruct((N, D), jnp.float32),
            jax.ShapeDtypeStruct((N, D), jnp.float32),
        ],
    )(y, ssum, ssq, g, be, w, b, dis)


def _bn_pool_body(y_ref, ssum_ref, ssq_ref, g_ref, be_ref, batch_ref,
                  segs_ref, cnt_ref):
    mu = ssum_ref[...] / N
    var = ssq_ref[...] / N - mu * mu
    rstd = lax.rsqrt(var + 1e-5)
    xn = (y_ref[...] - mu) * (rstd * g_ref[...]) + be_ref[...]
    seg_ids = lax.broadcasted_iota(jnp.int32, (1, NB), 1)
    oneh = jnp.where(batch_ref[...] == seg_ids, 1.0, 0.0)

    @pl.when(pl.program_id(0) == 0)
    def _():
        segs_ref[...] = jnp.zeros_like(segs_ref)
        cnt_ref[...] = jnp.zeros_like(cnt_ref)

    segs_ref[...] += lax.dot_general(oneh, xn, (((0,), (0,)), ((), ())),
                                     preferred_element_type=jnp.float32)
    ones_col = jnp.ones((R, 1), jnp.float32)
    cnt_ref[...] += lax.dot_general(oneh, ones_col, (((0,), (0,)), ((), ())),
                                    preferred_element_type=jnp.float32)


def _tc_bn_pool(y, ssum, ssq, g, be, batch2):
    return pl.pallas_call(
        _bn_pool_body,
        grid=(GRID,),
        in_specs=[
            pl.BlockSpec((R, D), lambda i: (i, 0)),
            pl.BlockSpec((1, D), lambda i: (0, 0)),
            pl.BlockSpec((1, D), lambda i: (0, 0)),
            pl.BlockSpec((1, D), lambda i: (0, 0)),
            pl.BlockSpec((1, D), lambda i: (0, 0)),
            pl.BlockSpec((R, 1), lambda i: (i, 0)),
        ],
        out_specs=[
            pl.BlockSpec((NB, D), lambda i: (0, 0)),
            pl.BlockSpec((NB, 1), lambda i: (0, 0)),
        ],
        out_shape=[
            jax.ShapeDtypeStruct((NB, D), jnp.float32),
            jax.ShapeDtypeStruct((NB, 1), jnp.float32),
        ],
    )(y, ssum, ssq, g, be, batch2)


def _head_body(segs_ref, cnt_ref, w1_ref, b1_ref, w2_ref, b2_ref, out_ref):
    pooled = segs_ref[...] / jnp.maximum(cnt_ref[...], 1.0)
    o = jnp.dot(pooled, w1_ref[...], preferred_element_type=jnp.float32)
    o = o + b1_ref[...]
    o = jnp.dot(o, w2_ref[...], preferred_element_type=jnp.float32)
    o = o + b2_ref[...]
    m = jnp.max(o, axis=-1, keepdims=True)
    lse = m + jnp.log(jnp.sum(jnp.exp(o - m), axis=-1, keepdims=True))
    out_ref[...] = o - lse


def _tc_head(segs, cnt, l1W, l1b, l2W, l2b):
    return pl.pallas_call(
        _head_body,
        out_shape=jax.ShapeDtypeStruct((NB, 4), jnp.float32),
    )(segs, cnt, l1W, l1b.reshape(1, -1), l2W, l2b.reshape(1, -1))


def kernel(x, edge_index, batch, W1, b1, g1, be1, W2, b2, g2, be2,
           l1W, l1b, l2W, l2b):
    src = edge_index[0]
    dst = edge_index[1]
    batch2 = batch.astype(jnp.int32).reshape(N, 1)
    zrows = jnp.zeros((ZR, D), jnp.float32)
    onesr = jnp.ones((K, D), jnp.float32)

    ppw = EPW - E // NW
    pad_ar = jnp.arange(ppw, dtype=src.dtype)
    src_pad = jnp.tile(pad_ar[None, :], (NW, 1))
    dst_pad = jnp.tile(N + (pad_ar % SINK)[None, :], (NW, 1))
    src3 = jnp.concatenate(
        [src.reshape(NW, -1), src_pad], axis=1).reshape(NW, CH, K)
    dst3 = jnp.concatenate(
        [dst.reshape(NW, -1), dst_pad], axis=1).reshape(NW, CH, K)
    sd = jnp.stack([src3, dst3], axis=2)

    degp = _sc_deg(dst3, onesr, zrows)

    Ws = [W1[0], W1[1], W1[2], W2[0], W2[1], W2[2]]
    bs = [b1[0].reshape(1, D), b1[1].reshape(1, D), b1[2].reshape(1, D),
          b2[0].reshape(1, D), b2[1].reshape(1, D), b2[2].reshape(1, D)]
    gs = [g1[0].reshape(1, D), g1[1].reshape(1, D), g1[2].reshape(1, D),
          g2[0].reshape(1, D), g2[1].reshape(1, D), g2[2].reshape(1, D)]
    bes = [be1[0].reshape(1, D), be1[1].reshape(1, D), be1[2].reshape(1, D),
           be2[0].reshape(1, D), be2[1].reshape(1, D), be2[2].reshape(1, D)]

    h, hp, dis = _tc_lead(x, Ws[0], bs[0], degp)
    segs = cnt = None
    for l in range(6):
        p = _sc_edge_agg(hp, sd, zrows)
        y, ssum, ssq = _tc_relu_stats(p, h, dis)
        if l < 5:
            h, hp = _tc_bn_matmul(y, ssum, ssq, gs[l], bes[l],
                                  Ws[l + 1], bs[l + 1], dis)
        else:
            segs, cnt = _tc_bn_pool(y, ssum, ssq, gs[l], bes[l], batch2)
    return _tc_head(segs, cnt, l1W, l1b, l2W, l2b)


# speedup vs baseline: 3.5207x; 1.0026x over previous
import functools

import jax
import jax.numpy as jnp
from jax import lax
from jax.experimental import pallas as pl
from jax.experimental.pallas import tpu as pltpu
from jax.experimental.pallas import tpu_sc as plsc

N = 10000
E = 320000
D = 128
NB = 64

NC = 2
NS = 16
NW = NC * NS
K = 128
CH = 81
NTRI = CH // 3
EPW = CH * K
EPAD = NW * EPW
SINK = 64
NA = N + SINK
RPS = 624
TAIL = N - NS * RPS
ZR = 208

R = 2000
GRID = N // R


@functools.cache
def _make_sc_edge_agg():
    mesh = plsc.VectorSubcoreMesh(core_axis_name="c", subcore_axis_name="s")

    @functools.partial(
        pl.kernel, mesh=mesh,
        out_type=jax.ShapeDtypeStruct((NC, N, D), jnp.float32),
        scratch_types=[
            pltpu.VMEM((2, K), jnp.int32),
            pltpu.VMEM((2, K), jnp.int32),
            pltpu.VMEM((2, K), jnp.int32),
            pltpu.VMEM((K, D), jnp.float32),
            pltpu.VMEM((K, D), jnp.float32),
            pltpu.VMEM((K, D), jnp.float32),
            pltpu.VMEM_SHARED((NA, D), jnp.float32),
            pltpu.SemaphoreType.DMA,
            pltpu.SemaphoreType.DMA,
        ],
    )
    def sc_edge_agg(hp_hbm, sd_hbm, zrows_hbm, out_hbm,
                    sd0, sd1, sd2, r0, r1, r2, acc, gsem, ssem):
        c = lax.axis_index("c")
        s = lax.axis_index("s")
        wid = c * NS + s
        for j in range(RPS // ZR):
            pltpu.sync_copy(zrows_hbm, acc.at[pl.ds(s * RPS + j * ZR, ZR)])

        @pl.when(s == 0)
        def _():
            pltpu.sync_copy(zrows_hbm.at[pl.ds(0, TAIL)],
                            acc.at[pl.ds(NS * RPS, TAIL)])
        plsc.subcore_barrier()

        bufs = ((sd0, r0), (sd1, r1), (sd2, r2))
        pltpu.sync_copy(sd_hbm.at[wid, 0], sd0)
        pltpu.async_copy(hp_hbm.at[sd0.at[0]], r0, gsem)
        pltpu.sync_copy(sd_hbm.at[wid, 1], sd1)
        pltpu.async_copy(hp_hbm.at[sd1.at[0]], r1, gsem)

        def tri(j, carry):
            for t in range(3):
                i = 3 * j + t
                sdX, rX = bufs[t]
                sdY, rY = bufs[(t + 2) % 3]
                pltpu.make_async_copy(hp_hbm.at[sdX.at[0]], rX, gsem).wait()
                pltpu.async_copy(rX, acc.at[sdX.at[1]], ssem, add=True)

                def drain_prev():
                    pltpu.make_async_copy(
                        rY, acc.at[sdY.at[1]], ssem).wait()

                def fetch_next():
                    pltpu.sync_copy(sd_hbm.at[wid, i + 2], sdY)
                    pltpu.async_copy(hp_hbm.at[sdY.at[0]], rY, gsem)

                if t == 0:
                    pl.when(j > 0)(drain_prev)
                    fetch_next()
                else:
                    drain_prev()
                    pl.when(j < NTRI - 1)(fetch_next)
            return carry

        lax.fori_loop(0, NTRI, tri, 0)
        pltpu.make_async_copy(r2, acc.at[sd2.at[1]], ssem).wait()
        plsc.subcore_barrier()
        pltpu.sync_copy(acc.at[pl.ds(s * RPS, RPS)],
                        out_hbm.at[c, pl.ds(s * RPS, RPS)])

        @pl.when(s == 0)
        def _():
            pltpu.sync_copy(acc.at[pl.ds(NS * RPS, TAIL)],
                            out_hbm.at[c, pl.ds(NS * RPS, TAIL)])

    return sc_edge_agg


def _sc_edge_agg(hp, sd, zrows):
    return _make_sc_edge_agg()(hp, sd, zrows)


@functools.cache
def _make_sc_deg():
    mesh = plsc.VectorSubcoreMesh(core_axis_name="c", subcore_axis_name="s")

    @functools.partial(
        pl.kernel, mesh=mesh,
        out_type=jax.ShapeDtypeStruct((NC, N, D), jnp.float32),
        scratch_types=[
            pltpu.VMEM((CH, K), jnp.int32),
            pltpu.VMEM((K, D), jnp.float32),
            pltpu.VMEM_SHARED((NA, D), jnp.float32),
            pltpu.SemaphoreType.DMA,
        ],
    )
    def sc_deg(dst3_hbm, ones_hbm, zrows_hbm, out_hbm, dsts_v, ones_v, acc,
               ssem):
        c = lax.axis_index("c")
        s = lax.axis_index("s")
        wid = c * NS + s
        pltpu.sync_copy(dst3_hbm.at[wid], dsts_v)
        for j in range(RPS // ZR):
            pltpu.sync_copy(zrows_hbm, acc.at[pl.ds(s * RPS + j * ZR, ZR)])

        @pl.when(s == 0)
        def _():
            pltpu.sync_copy(zrows_hbm.at[pl.ds(0, TAIL)],
                            acc.at[pl.ds(NS * RPS, TAIL)])
        pltpu.sync_copy(ones_hbm, ones_v)
        plsc.subcore_barrier()

        def step(i, carry):
            pltpu.async_copy(ones_v, acc.at[dsts_v.at[i]], ssem, add=True)

            @pl.when(i >= 4)
            def _():
                pltpu.make_async_copy(
                    ones_v, acc.at[dsts_v.at[0]], ssem).wait()
            return carry

        lax.fori_loop(0, CH, step, 0)
        for _ in range(4):
            pltpu.make_async_copy(ones_v, acc.at[dsts_v.at[0]], ssem).wait()
        plsc.subcore_barrier()
        pltpu.sync_copy(acc.at[pl.ds(s * RPS, RPS)],
                        out_hbm.at[c, pl.ds(s * RPS, RPS)])

        @pl.when(s == 0)
        def _():
            pltpu.sync_copy(acc.at[pl.ds(NS * RPS, TAIL)],
                            out_hbm.at[c, pl.ds(NS * RPS, TAIL)])

    return sc_deg


def _sc_deg(dst3, onesr, zrows):
    return _make_sc_deg()(dst3, onesr, zrows)


def _lead_body(x_ref, w_ref, b_ref, degp_ref, h_ref, hp_ref, dis_ref):
    deg = degp_ref[0, :, 0:1] + degp_ref[1, :, 0:1] + 1.0
    dis = lax.rsqrt(deg)
    h = jnp.dot(x_ref[...], w_ref[...], preferred_element_type=jnp.float32)
    h = h + b_ref[...]
    h_ref[...] = h
    hp_ref[...] = h * dis
    dis_ref[...] = dis


def _tc_lead(x, w, b, degp):
    return pl.pallas_call(
        _lead_body,
        grid=(GRID,),
        in_specs=[
            pl.BlockSpec((R, D), lambda i: (i, 0)),
            pl.BlockSpec((D, D), lambda i: (0, 0)),
            pl.BlockSpec((1, D), lambda i: (0, 0)),
            pl.BlockSpec((NC, R, D), lambda i: (0, i, 0)),
        ],
        out_specs=[
            pl.BlockSpec((R, D), lambda i: (i, 0)),
            pl.BlockSpec((R, D), lambda i: (i, 0)),
            pl.BlockSpec((R, 1), lambda i: (i, 0)),
        ],
        out_shape=[
            jax.ShapeDtypeStruct((N, D), jnp.float32),
            jax.ShapeDtypeStruct((N, D), jnp.float32),
            jax.ShapeDtypeStruct((N, 1), jnp.float32),
        ],
    )(x, w, b, degp)


def _relu_stats_body(p_ref, h_ref, dis_ref, y_ref, ssum_ref, ssq_ref):
    dis = dis_ref[...]
    agg = dis * (p_ref[0] + p_ref[1]) + (dis * dis) * h_ref[...]
    y = jnp.maximum(agg, 0.0)
    y_ref[...] = y

    @pl.when(pl.program_id(0) == 0)
    def _():
        ssum_ref[...] = jnp.zeros_like(ssum_ref)
        ssq_ref[...] = jnp.zeros_like(ssq_ref)

    ssum_ref[...] += jnp.sum(y, axis=0, keepdims=True)
    ssq_ref[...] += jnp.sum(y * y, axis=0, keepdims=True)


def _tc_relu_stats(p, h, dis):
    return pl.pallas_call(
        _relu_stats_body,
        grid=(GRID,),
        in_specs=[
            pl.BlockSpec((NC, R, D), lambda i: (0, i, 0)),
            pl.BlockSpec((R, D), lambda i: (i, 0)),
            pl.BlockSpec((R, 1), lambda i: (i, 0)),
        ],
        out_specs=[
            pl.BlockSpec((R, D), lambda i: (i, 0)),
            pl.BlockSpec((1, D), lambda i: (0, 0)),
            pl.BlockSpec((1, D), lambda i: (0, 0)),
        ],
        out_shape=[
            jax.ShapeDtypeStruct((N, D), jnp.float32),
            jax.ShapeDtypeStruct((1, D), jnp.float32),
            jax.ShapeDtypeStruct((1, D), jnp.float32),
        ],
    )(p, h, dis)


def _bn_matmul_body(y_ref, ssum_ref, ssq_ref, g_ref, be_ref, w_ref, b_ref,
                    dis_ref, h_ref, hp_ref):
    mu = ssum_ref[...] / N
    var = ssq_ref[...] / N - mu * mu
    rstd = lax.rsqrt(var + 1e-5)
    xn = (y_ref[...] - mu) * (rstd * g_ref[...]) + be_ref[...]
    h = jnp.dot(xn, w_ref[...], preferred_element_type=jnp.float32)
    h = h + b_ref[...]
    h_ref[...] = h
    hp_ref[...] = h * dis_ref[...]


def _tc_bn_matmul(y, ssum, ssq, g, be, w, b, dis):
    return pl.pallas_call(
        _bn_matmul_body,
        grid=(GRID,),
        in_specs=[
            pl.BlockSpec((R, D), lambda i: (i, 0)),
            pl.BlockSpec((1, D), lambda i: (0, 0)),
            pl.BlockSpec((1, D), lambda i: (0, 0)),
            pl.BlockSpec((1, D), lambda i: (0, 0)),
            pl.BlockSpec((1, D), lambda i: (0, 0)),
            pl.BlockSpec((D, D), lambda i: (0, 0)),
            pl.BlockSpec((1, D), lambda i: (0, 0)),
            pl.BlockSpec((R, 1), lambda i: (i, 0)),
        ],
        out_specs=[
            pl.BlockSpec((R, D), lambda i: (i, 0)),
            pl.BlockSpec((R, D), lambda i: (i, 0)),
        ],
        out_shape=[
            jax.ShapeDtypeStruct((N, D), jnp.float32),
            jax.ShapeDtypeStruct((N, D), jnp.float32),
        ],
    )(y, ssum, ssq, g, be, w, b, dis)


def _bn_pool_body(y_ref, ssum_ref, ssq_ref, g_ref, be_ref, batch_ref,
                  segs_ref, cnt_ref):
    mu = ssum_ref[...] / N
    var = ssq_ref[...] / N - mu * mu
    rstd = lax.rsqrt(var + 1e-5)
    xn = (y_ref[...] - mu) * (rstd * g_ref[...]) + be_ref[...]
    seg_ids = lax.broadcasted_iota(jnp.int32, (1, NB), 1)
    oneh = jnp.where(batch_ref[...] == seg_ids, 1.0, 0.0)

    @pl.when(pl.program_id(0) == 0)
    def _():
        segs_ref[...] = jnp.zeros_like(segs_ref)
        cnt_ref[...] = jnp.zeros_like(cnt_ref)

    segs_ref[...] += lax.dot_general(oneh, xn, (((0,), (0,)), ((), ())),
                                     preferred_element_type=jnp.float32)
    ones_col = jnp.ones((R, 1), jnp.float32)
    cnt_ref[...] += lax.dot_general(oneh, ones_col, (((0,), (0,)), ((), ())),
                                    preferred_element_type=jnp.float32)


def _tc_bn_pool(y, ssum, ssq, g, be, batch2):
    return pl.pallas_call(
        _bn_pool_body,
        grid=(GRID,),
        in_specs=[
            pl.BlockSpec((R, D), lambda i: (i, 0)),
            pl.BlockSpec((1, D), lambda i: (0, 0)),
            pl.BlockSpec((1, D), lambda i: (0, 0)),
            pl.BlockSpec((1, D), lambda i: (0, 0)),
            pl.BlockSpec((1, D), lambda i: (0, 0)),
            pl.BlockSpec((R, 1), lambda i: (i, 0)),
        ],
        out_specs=[
            pl.BlockSpec((NB, D), lambda i: (0, 0)),
            pl.BlockSpec((NB, 1), lambda i: (0, 0)),
        ],
        out_shape=[
            jax.ShapeDtypeStruct((NB, D), jnp.float32),
            jax.ShapeDtypeStruct((NB, 1), jnp.float32),
        ],
    )(y, ssum, ssq, g, be, batch2)


def _head_body(segs_ref, cnt_ref, w1_ref, b1_ref, w2_ref, b2_ref, out_ref):
    pooled = segs_ref[...] / jnp.maximum(cnt_ref[...], 1.0)
    o = jnp.dot(pooled, w1_ref[...], preferred_element_type=jnp.float32)
    o = o + b1_ref[...]
    o = jnp.dot(o, w2_ref[...], preferred_element_type=jnp.float32)
    o = o + b2_ref[...]
    m = jnp.max(o, axis=-1, keepdims=True)
    lse = m + jnp.log(jnp.sum(jnp.exp(o - m), axis=-1, keepdims=True))
    out_ref[...] = o - lse


def _tc_head(segs, cnt, l1W, l1b, l2W, l2b):
    return pl.pallas_call(
        _head_body,
        out_shape=jax.ShapeDtypeStruct((NB, 4), jnp.float32),
    )(segs, cnt, l1W, l1b.reshape(1, -1), l2W, l2b.reshape(1, -1))


def kernel(x, edge_index, batch, W1, b1, g1, be1, W2, b2, g2, be2,
           l1W, l1b, l2W, l2b):
    src = edge_index[0]
    dst = edge_index[1]
    batch2 = batch.astype(jnp.int32).reshape(N, 1)
    zrows = jnp.zeros((ZR, D), jnp.float32)
    onesr = jnp.ones((K, D), jnp.float32)

    ppw = EPW - E // NW
    pad_ar = jnp.arange(ppw, dtype=src.dtype)
    src_pad = jnp.tile(pad_ar[None, :], (NW, 1))
    dst_pad = jnp.tile(N + (pad_ar % SINK)[None, :], (NW, 1))
    src3 = jnp.concatenate(
        [src.reshape(NW, -1), src_pad], axis=1).reshape(NW, CH, K)
    dst3 = jnp.concatenate(
        [dst.reshape(NW, -1), dst_pad], axis=1).reshape(NW, CH, K)
    sd = jnp.stack([src3, dst3], axis=2)

    degp = _sc_deg(dst3, onesr, zrows)

    Ws = [W1[0], W1[1], W1[2], W2[0], W2[1], W2[2]]
    bs = [b1[0].reshape(1, D), b1[1].reshape(1, D), b1[2].reshape(1, D),
          b2[0].reshape(1, D), b2[1].reshape(1, D), b2[2].reshape(1, D)]
    gs = [g1[0].reshape(1, D), g1[1].reshape(1, D), g1[2].reshape(1, D),
          g2[0].reshape(1, D), g2[1].reshape(1, D), g2[2].reshape(1, D)]
    bes = [be1[0].reshape(1, D), be1[1].reshape(1, D), be1[2].reshape(1, D),
           be2[0].reshape(1, D), be2[1].reshape(1, D), be2[2].reshape(1, D)]

    h, hp, dis = _tc_lead(x, Ws[0], bs[0], degp)
    segs = cnt = None
    for l in range(6):
        p = _sc_edge_agg(hp, sd, zrows)
        y, ssum, ssq = _tc_relu_stats(p, h, dis)
        if l < 5:
            h, hp = _tc_bn_matmul(y, ssum, ssq, gs[l], bes[l],
                                  Ws[l + 1], bs[l + 1], dis)
        else:
            segs, cnt = _tc_bn_pool(y, ssum, ssq, gs[l], bes[l], batch2)
    return _tc_head(segs, cnt, l1W, l1b, l2W, l2b)
